# baseline (speedup 1.0000x reference)
import functools

import jax
import jax.numpy as jnp
from jax import lax
from jax.experimental import pallas as pl
from jax.experimental.pallas import tpu as pltpu
from jax.experimental.pallas import tpu_sc as plsc

_N = 10000
_D = 256
_H = 4
_DK = 64
_I = 5

_HD = _H * _DK
_R = _I * _N
_NW = 32
_PER_W = 1600
_TPAD = _NW * _PER_W
_CHUNK = 80
_NCHUNK = _PER_W // _CHUNK

_BN = 400


def _mm_body(x_ref, wq_ref, w2_ref, q_ref, z_ref):
    xb = x_ref[...].astype(jnp.bfloat16)
    for h in range(_H):
        q_ref[h] = jnp.dot(xb, wq_ref[h].astype(jnp.bfloat16),
                           preferred_element_type=jnp.float32)
    for i in range(_I):
        z_ref[i] = jnp.dot(xb, w2_ref[i].astype(jnp.bfloat16),
                           preferred_element_type=jnp.float32)


def _tc_proj(x, wq, w2):
    return pl.pallas_call(
        _mm_body,
        grid=(_N // _BN,),
        in_specs=[
            pl.BlockSpec((_BN, _D), lambda n: (n, 0)),
            pl.BlockSpec((_H, _D, _DK), lambda n: (0, 0, 0)),
            pl.BlockSpec((_I, _D, _HD), lambda n: (0, 0, 0)),
        ],
        out_specs=[
            pl.BlockSpec((_H, _BN, _DK), lambda n: (0, n, 0)),
            pl.BlockSpec((_I, _BN, _HD), lambda n: (0, n, 0)),
        ],
        out_shape=[
            jax.ShapeDtypeStruct((_H, _N, _DK), jnp.float32),
            jax.ShapeDtypeStruct((_I, _N, _HD), jnp.float32),
        ],
    )(x, wq, w2)


@functools.lru_cache(maxsize=None)
def _make_sc_gather():
    @functools.partial(
        pl.kernel,
        mesh=plsc.VectorSubcoreMesh(core_axis_name="c", subcore_axis_name="s"),
        out_type=jax.ShapeDtypeStruct((_H, _R, _DK), jnp.float32),
        scratch_types=[
            pltpu.VMEM((_CHUNK,), jnp.int32),
            pltpu.VMEM((_CHUNK, _HD), jnp.float32),
            pltpu.SemaphoreType.DMA,
        ],
        compiler_params=pltpu.CompilerParams(use_tc_tiling_on_sc=False),
    )
    def _sc_gather(table_hbm, idx_hbm, out_hbm, idx_v, rows_v, sem):
        wid = lax.axis_index("s") * 2 + lax.axis_index("c")
        base = wid * _PER_W

        def body(c, carry):
            off = pl.multiple_of(base + c * _CHUNK, 16)

            @pl.when(off < _R)
            def _():
                pltpu.sync_copy(idx_hbm.at[pl.ds(off, _CHUNK)], idx_v)
                pltpu.async_copy(table_hbm.at[idx_v], rows_v, sem).wait()
                for h in range(_H):
                    pltpu.sync_copy(
                        rows_v.at[:, pl.ds(h * _DK, _DK)],
                        out_hbm.at[h, pl.ds(off, _CHUNK)])

            return carry

        lax.fori_loop(0, _NCHUNK, body, 0)

    return _sc_gather


def kernel(x, adjacency_matrix, w_q, w_k, w_v):
    del w_v
    w2 = w_k.transpose(1, 2, 0, 3).reshape(_I, _D, _HD)
    q, z = _tc_proj(x, w_q, w2)

    offs = (jnp.arange(_I, dtype=jnp.int32) * _N)[:, None]
    idxg = (adjacency_matrix + offs).reshape(-1)
    idxg = jnp.concatenate(
        [idxg, jnp.zeros((_TPAD - _R,), dtype=jnp.int32)])

    kflat = _make_sc_gather()(z.reshape(_R, _HD), idxg)
    return (q, kflat.reshape(_H, _I, _N, _DK))


# speedup vs baseline: 1.1265x; 1.1265x over previous
import functools

import jax
import jax.numpy as jnp
from jax import lax
from jax.experimental import pallas as pl
from jax.experimental.pallas import tpu as pltpu
from jax.experimental.pallas import tpu_sc as plsc

_N = 10000
_D = 256
_H = 4
_DK = 64
_I = 5

_R = _I * _N
_NW = 32
_PER_W = 1600
_TPAD = _NW * _PER_W
_CHUNK = 80
_NCHUNK = _PER_W // _CHUNK

_BN = 400


@functools.lru_cache(maxsize=None)
def _make_sc_gather():
    @functools.partial(
        pl.kernel,
        mesh=plsc.VectorSubcoreMesh(core_axis_name="c", subcore_axis_name="s"),
        out_type=jax.ShapeDtypeStruct((_R, _D), jnp.float32),
        scratch_types=[
            pltpu.VMEM((_CHUNK,), jnp.int32),
            pltpu.VMEM((_CHUNK, _D), jnp.float32),
            pltpu.SemaphoreType.DMA,
        ],
    )
    def _sc_gather(table_hbm, idx_hbm, out_hbm, idx_v, rows_v, sem):
        wid = lax.axis_index("s") * 2 + lax.axis_index("c")
        base = wid * _PER_W

        def body(c, carry):
            off = pl.multiple_of(base + c * _CHUNK, 16)

            @pl.when(off < _R)
            def _():
                pltpu.sync_copy(idx_hbm.at[pl.ds(off, _CHUNK)], idx_v)
                pltpu.async_copy(table_hbm.at[idx_v], rows_v, sem).wait()
                pltpu.sync_copy(rows_v, out_hbm.at[pl.ds(off, _CHUNK)])

            return carry

        lax.fori_loop(0, _NCHUNK, body, 0)

    return _sc_gather


def _q_body(x_ref, wq_ref, q_ref):
    xb = x_ref[...].astype(jnp.bfloat16)
    for h in range(_H):
        q_ref[h] = jnp.dot(xb, wq_ref[h].astype(jnp.bfloat16),
                           preferred_element_type=jnp.float32)


def _tc_q(x, wq):
    return pl.pallas_call(
        _q_body,
        grid=(_N // _BN,),
        in_specs=[
            pl.BlockSpec((_BN, _D), lambda n: (n, 0)),
            pl.BlockSpec((_H, _D, _DK), lambda n: (0, 0, 0)),
        ],
        out_specs=pl.BlockSpec((_H, _BN, _DK), lambda n: (0, n, 0)),
        out_shape=jax.ShapeDtypeStruct((_H, _N, _DK), jnp.float32),
    )(x, wq)


def _k_body(xn_ref, wk_ref, k_ref):
    xb = xn_ref[0].astype(jnp.bfloat16)
    for h in range(_H):
        k_ref[h, 0] = jnp.dot(xb, wk_ref[h, 0].astype(jnp.bfloat16),
                              preferred_element_type=jnp.float32)


def _tc_k(xn, wk):
    return pl.pallas_call(
        _k_body,
        grid=(_I, _N // _BN),
        in_specs=[
            pl.BlockSpec((1, _BN, _D), lambda i, n: (i, n, 0)),
            pl.BlockSpec((_H, 1, _D, _DK), lambda i, n: (0, i, 0, 0)),
        ],
        out_specs=pl.BlockSpec((_H, 1, _BN, _DK), lambda i, n: (0, i, n, 0)),
        out_shape=jax.ShapeDtypeStruct((_H, _I, _N, _DK), jnp.float32),
    )(xn, wk)


def kernel(x, adjacency_matrix, w_q, w_k, w_v):
    del w_v
    idx = adjacency_matrix.reshape(-1)
    idx = jnp.concatenate([idx, jnp.zeros((_TPAD - _R,), dtype=jnp.int32)])
    xn = _make_sc_gather()(x, idx)
    q = _tc_q(x, w_q)
    k = _tc_k(xn.reshape(_I, _N, _D), w_k)
    return (q, k)


# speedup vs baseline: 2.1947x; 1.9482x over previous
import functools

import jax
import jax.numpy as jnp
from jax import lax
from jax.experimental import pallas as pl
from jax.experimental.pallas import tpu as pltpu
from jax.experimental.pallas import tpu_sc as plsc

_N = 10000
_D = 256
_H = 4
_DK = 64
_I = 5

_R = _I * _N
_NW = 32
_PER_W = 1600
_TPAD = _NW * _PER_W
_CHUNK = 80
_NCHUNK = _PER_W // _CHUNK

_BN = 400

_NT_DIMS = (((1,), (1,)), ((), ()))


@functools.lru_cache(maxsize=None)
def _make_sc_gather():
    @functools.partial(
        pl.kernel,
        mesh=plsc.VectorSubcoreMesh(core_axis_name="c", subcore_axis_name="s"),
        out_type=jax.ShapeDtypeStruct((_R, _D), jnp.float32),
        scratch_types=[
            pltpu.VMEM((_CHUNK,), jnp.int32),
            pltpu.VMEM((_CHUNK, _D), jnp.float32),
            pltpu.SemaphoreType.DMA,
        ],
    )
    def _sc_gather(table_hbm, idx_hbm, out_hbm, idx_v, rows_v, sem):
        wid = lax.axis_index("s") * 2 + lax.axis_index("c")
        base = wid * _PER_W

        def body(c, carry):
            off = pl.multiple_of(base + c * _CHUNK, 16)

            @pl.when(off < _R)
            def _():
                pltpu.sync_copy(idx_hbm.at[pl.ds(off, _CHUNK)], idx_v)
                pltpu.async_copy(table_hbm.at[idx_v], rows_v, sem).wait()
                pltpu.sync_copy(rows_v, out_hbm.at[pl.ds(off, _CHUNK)])

            return carry

        lax.fori_loop(0, _NCHUNK, body, 0)

    return _sc_gather


def _q_body(x_ref, wqt_ref, q_ref):
    q_ref[0] = lax.dot_general(
        wqt_ref[0].astype(jnp.bfloat16), x_ref[...].astype(jnp.bfloat16),
        _NT_DIMS, preferred_element_type=jnp.float32)


def _tc_q(x, wqt):
    return pl.pallas_call(
        _q_body,
        grid=(_H,),
        in_specs=[
            pl.BlockSpec((_N, _D), lambda h: (0, 0)),
            pl.BlockSpec((1, _DK, _D), lambda h: (h, 0, 0)),
        ],
        out_specs=pl.BlockSpec((1, _DK, _N), lambda h: (h, 0, 0)),
        out_shape=jax.ShapeDtypeStruct((_H, _DK, _N), jnp.float32),
    )(x, wqt)


def _k_body(xn_ref, wkt_ref, k_ref):
    k_ref[0, 0] = lax.dot_general(
        wkt_ref[0, 0].astype(jnp.bfloat16), xn_ref[0].astype(jnp.bfloat16),
        _NT_DIMS, preferred_element_type=jnp.float32)


def _tc_k(xn, wkt):
    return pl.pallas_call(
        _k_body,
        grid=(_I, _H),
        in_specs=[
            pl.BlockSpec((1, _N, _D), lambda i, h: (i, 0, 0)),
            pl.BlockSpec((1, 1, _DK, _D), lambda i, h: (h, i, 0, 0)),
        ],
        out_specs=pl.BlockSpec((1, 1, _DK, _N), lambda i, h: (h, i, 0, 0)),
        out_shape=jax.ShapeDtypeStruct((_H, _I, _DK, _N), jnp.float32),
    )(xn, wkt)


def kernel(x, adjacency_matrix, w_q, w_k, w_v):
    del w_v
    idx = adjacency_matrix.reshape(-1)
    idx = jnp.concatenate([idx, jnp.zeros((_TPAD - _R,), dtype=jnp.int32)])
    xn = _make_sc_gather()(x, idx)
    qt = _tc_q(x, w_q.transpose(0, 2, 1))
    kt = _tc_k(xn.reshape(_I, _N, _D), w_k.transpose(0, 1, 3, 2))
    return (qt.transpose(0, 2, 1), kt.transpose(0, 1, 3, 2))


# speedup vs baseline: 2.4694x; 1.1252x over previous
import functools

import jax
import jax.numpy as jnp
from jax import lax
from jax.experimental import pallas as pl
from jax.experimental.pallas import tpu as pltpu
from jax.experimental.pallas import tpu_sc as plsc

_N = 10000
_D = 256
_H = 4
_DK = 64
_I = 5

_NW = 32
_PER_W = 320
_NPAD = _NW * _PER_W
_CHUNK = 80
_NCHUNK = _PER_W // _CHUNK

_NT_DIMS = (((1,), (1,)), ((), ()))


@functools.lru_cache(maxsize=None)
def _make_sc_gather():
    @functools.partial(
        pl.kernel,
        mesh=plsc.VectorSubcoreMesh(core_axis_name="c", subcore_axis_name="s"),
        out_type=jax.ShapeDtypeStruct((_N, _D), jnp.float32),
        scratch_types=[
            pltpu.VMEM((_CHUNK,), jnp.int32),
            pltpu.VMEM((_CHUNK, _D), jnp.float32),
            pltpu.SemaphoreType.DMA,
        ],
    )
    def _sc_gather(table_hbm, idx_hbm, out_hbm, idx_v, rows_v, sem):
        wid = lax.axis_index("s") * 2 + lax.axis_index("c")
        base = wid * _PER_W

        def body(c, carry):
            off = pl.multiple_of(base + c * _CHUNK, 16)

            @pl.when(off < _N)
            def _():
                pltpu.sync_copy(idx_hbm.at[pl.ds(off, _CHUNK)], idx_v)
                pltpu.async_copy(table_hbm.at[idx_v], rows_v, sem).wait()
                pltpu.sync_copy(rows_v, out_hbm.at[pl.ds(off, _CHUNK)])

            return carry

        lax.fori_loop(0, _NCHUNK, body, 0)

    return _sc_gather


def _q_body(x_ref, wqt_ref, q_ref):
    q_ref[0] = lax.dot_general(
        wqt_ref[0].astype(jnp.bfloat16), x_ref[...].astype(jnp.bfloat16),
        _NT_DIMS, preferred_element_type=jnp.float32)


def _tc_q(x, wqt):
    return pl.pallas_call(
        _q_body,
        grid=(_H,),
        in_specs=[
            pl.BlockSpec((_N, _D), lambda h: (0, 0)),
            pl.BlockSpec((1, _DK, _D), lambda h: (h, 0, 0)),
        ],
        out_specs=pl.BlockSpec((1, _DK, _N), lambda h: (h, 0, 0)),
        out_shape=jax.ShapeDtypeStruct((_H, _DK, _N), jnp.float32),
    )(x, wqt)


def _k_body_first(xn_ref, wkt_ref, k_ref):
    k_ref[0, 0] = lax.dot_general(
        wkt_ref[0, 0].astype(jnp.bfloat16), xn_ref[...].astype(jnp.bfloat16),
        _NT_DIMS, preferred_element_type=jnp.float32)


def _k_body_acc(kin_ref, xn_ref, wkt_ref, k_ref):
    del kin_ref
    k_ref[0, 0] = lax.dot_general(
        wkt_ref[0, 0].astype(jnp.bfloat16), xn_ref[...].astype(jnp.bfloat16),
        _NT_DIMS, preferred_element_type=jnp.float32)


def _tc_k_slot(i, kt, xn_i, wkt):
    out_spec = pl.BlockSpec((1, 1, _DK, _N), lambda h: (h, i, 0, 0))
    out_shape = jax.ShapeDtypeStruct((_H, _I, _DK, _N), jnp.float32)
    common = [
        pl.BlockSpec((_N, _D), lambda h: (0, 0)),
        pl.BlockSpec((1, 1, _DK, _D), lambda h, _i=i: (h, _i, 0, 0)),
    ]
    if kt is None:
        return pl.pallas_call(
            _k_body_first,
            grid=(_H,),
            in_specs=common,
            out_specs=out_spec,
            out_shape=out_shape,
        )(xn_i, wkt)
    return pl.pallas_call(
        _k_body_acc,
        grid=(_H,),
        in_specs=[pl.BlockSpec(memory_space=pl.ANY)] + common,
        out_specs=out_spec,
        out_shape=out_shape,
        input_output_aliases={0: 0},
    )(kt, xn_i, wkt)


def kernel(x, adjacency_matrix, w_q, w_k, w_v):
    del w_v
    idx = jnp.pad(adjacency_matrix, ((0, 0), (0, _NPAD - _N)))
    gather = _make_sc_gather()
    xns = [gather(x, idx[i]) for i in range(_I)]
    qt = _tc_q(x, w_q.transpose(0, 2, 1))
    wkt = w_k.transpose(0, 1, 3, 2)
    kt = None
    for i in range(_I):
        kt = _tc_k_slot(i, kt, xns[i], wkt)
    return (qt.transpose(0, 2, 1), kt.transpose(0, 1, 3, 2))


# speedup vs baseline: 2.6607x; 1.0775x over previous
import functools

import jax
import jax.numpy as jnp
from jax import lax
from jax.experimental import pallas as pl
from jax.experimental.pallas import tpu as pltpu
from jax.experimental.pallas import tpu_sc as plsc

_N = 10000
_D = 256
_H = 4
_DK = 64
_I = 5

_NW = 32
_PER_W = 320
_NPAD = _NW * _PER_W
_CHUNK = 80
_NCHUNK = _PER_W // _CHUNK

_NT_DIMS = (((1,), (1,)), ((), ()))


@functools.lru_cache(maxsize=None)
def _make_sc_gather():
    @functools.partial(
        pl.kernel,
        mesh=plsc.VectorSubcoreMesh(core_axis_name="c", subcore_axis_name="s"),
        out_type=jax.ShapeDtypeStruct((_N, _D), jnp.float32),
        scratch_types=[
            pltpu.VMEM((_PER_W,), jnp.int32),
            pltpu.VMEM((_CHUNK, _D), jnp.float32),
            pltpu.VMEM((_CHUNK, _D), jnp.float32),
            pltpu.SemaphoreType.DMA,
            pltpu.SemaphoreType.DMA,
            pltpu.SemaphoreType.DMA,
            pltpu.SemaphoreType.DMA,
        ],
    )
    def _sc_gather(table_hbm, idx_hbm, out_hbm, idx_v,
                   rows_a, rows_b, gs_a, gs_b, ws_a, ws_b):
        wid = lax.axis_index("s") * 2 + lax.axis_index("c")
        base = wid * _PER_W
        pltpu.sync_copy(idx_hbm.at[pl.ds(base, _PER_W)], idx_v)

        rows = (rows_a, rows_b)
        gs = (gs_a, gs_b)
        ws = (ws_a, ws_b)
        offs = [pl.multiple_of(base + c * _CHUNK, 16) for c in range(_NCHUNK)]
        conds = [offs[c] < _N for c in range(_NCHUNK)]

        def gstart(c):
            b = c % 2
            pltpu.async_copy(
                table_hbm.at[idx_v.at[pl.ds(c * _CHUNK, _CHUNK)]],
                rows[b], gs[b])

        def gwait(c):
            b = c % 2
            pltpu.make_async_copy(
                table_hbm.at[idx_v.at[pl.ds(c * _CHUNK, _CHUNK)]],
                rows[b], gs[b]).wait()

        def wstart(c):
            b = c % 2
            pltpu.async_copy(rows[b], out_hbm.at[pl.ds(offs[c], _CHUNK)],
                             ws[b])

        def wwait(c):
            b = c % 2
            pltpu.make_async_copy(rows[b], out_hbm.at[pl.ds(offs[c], _CHUNK)],
                                  ws[b]).wait()

        for c in range(_NCHUNK):
            if c >= 2:
                @pl.when(conds[c - 2])
                def _(c=c):
                    wwait(c - 2)

            @pl.when(conds[c])
            def _(c=c):
                gstart(c)

            @pl.when(conds[c])
            def _(c=c):
                gwait(c)
                wstart(c)
        for c in range(max(_NCHUNK - 2, 0), _NCHUNK):
            @pl.when(conds[c])
            def _(c=c):
                wwait(c)

    return _sc_gather


def _q_body(x_ref, wqt_ref, q_ref):
    q_ref[0] = lax.dot_general(
        wqt_ref[0].astype(jnp.bfloat16), x_ref[...].astype(jnp.bfloat16),
        _NT_DIMS, preferred_element_type=jnp.float32)


def _tc_q(x, wqt):
    return pl.pallas_call(
        _q_body,
        grid=(_H,),
        in_specs=[
            pl.BlockSpec((_N, _D), lambda h: (0, 0)),
            pl.BlockSpec((1, _DK, _D), lambda h: (h, 0, 0)),
        ],
        out_specs=pl.BlockSpec((1, _DK, _N), lambda h: (h, 0, 0)),
        out_shape=jax.ShapeDtypeStruct((_H, _DK, _N), jnp.float32),
    )(x, wqt)


def _k_body_first(xn_ref, wkt_ref, k_ref):
    k_ref[0, 0] = lax.dot_general(
        wkt_ref[0, 0].astype(jnp.bfloat16), xn_ref[...].astype(jnp.bfloat16),
        _NT_DIMS, preferred_element_type=jnp.float32)


def _k_body_acc(kin_ref, xn_ref, wkt_ref, k_ref):
    del kin_ref
    k_ref[0, 0] = lax.dot_general(
        wkt_ref[0, 0].astype(jnp.bfloat16), xn_ref[...].astype(jnp.bfloat16),
        _NT_DIMS, preferred_element_type=jnp.float32)


def _tc_k_slot(i, kt, xn_i, wkt):
    out_spec = pl.BlockSpec((1, 1, _DK, _N), lambda h: (h, i, 0, 0))
    out_shape = jax.ShapeDtypeStruct((_H, _I, _DK, _N), jnp.float32)
    common = [
        pl.BlockSpec((_N, _D), lambda h: (0, 0)),
        pl.BlockSpec((1, 1, _DK, _D), lambda h, _i=i: (h, _i, 0, 0)),
    ]
    if kt is None:
        return pl.pallas_call(
            _k_body_first,
            grid=(_H,),
            in_specs=common,
            out_specs=out_spec,
            out_shape=out_shape,
        )(xn_i, wkt)
    return pl.pallas_call(
        _k_body_acc,
        grid=(_H,),
        in_specs=[pl.BlockSpec(memory_space=pl.ANY)] + common,
        out_specs=out_spec,
        out_shape=out_shape,
        input_output_aliases={0: 0},
    )(kt, xn_i, wkt)


def kernel(x, adjacency_matrix, w_q, w_k, w_v):
    del w_v
    idx = jnp.pad(adjacency_matrix, ((0, 0), (0, _NPAD - _N)))
    gather = _make_sc_gather()
    xns = [gather(x, idx[i]) for i in range(_I)]
    qt = _tc_q(x, w_q.transpose(0, 2, 1))
    wkt = w_k.transpose(0, 1, 3, 2)
    kt = None
    for i in range(_I):
        kt = _tc_k_slot(i, kt, xns[i], wkt)
    return (qt.transpose(0, 2, 1), kt.transpose(0, 1, 3, 2))
